# Initial kernel scaffold; baseline (speedup 1.0000x reference)
import jax
import jax.numpy as jnp
from jax.experimental import pallas as pl


def kernel(h, W, a, edge_mask):
    raise NotImplementedError("write your pallas kernel here")


# speedup vs baseline: 96.8133x; 96.8133x over previous
import functools

import jax
import jax.numpy as jnp
from jax.experimental import pallas as pl
from jax.experimental.pallas import tpu as pltpu


def _body(rows, h_ref, w_ref, ac_ref, m_ref, o_ref, s1_ref, s2_ref):
    i = pl.program_id(0)

    @pl.when(i == 0)
    def _():
        wa = jnp.dot(w_ref[...], ac_ref[...], preferred_element_type=jnp.float32)
        s = jnp.dot(h_ref[...], wa, preferred_element_type=jnp.float32)
        s1_ref[...] = s[:, 0:1]
        s2_ref[...] = jnp.transpose(s[:, 1:2])

    n = s2_ref.shape[1]
    m = m_ref[...]
    s1 = s1_ref[pl.ds(i * rows, rows), :]
    z = s1 + s2_ref[...]
    z = jnp.where(z >= 0.0, z, 0.1 * z)
    e = jnp.where(m != 0.0, jnp.exp(z), 0.0)
    rs = jnp.sum(e, axis=1, keepdims=True)
    pos = rs == 0.0
    col = jax.lax.broadcasted_iota(jnp.int32, (rows, n), 1)
    row = i * rows + jax.lax.broadcasted_iota(jnp.int32, (rows, n), 0)
    e = jnp.where((col == row) & pos, 1.0, e)
    rs = jnp.where(pos, 1.0, rs)
    o_ref[...] = e / rs


@jax.jit
def kernel(h, W, a, edge_mask):
    n, fin = h.shape
    hh = W.shape[1]
    rows = 256
    a_cols = a.reshape(2, hh).T

    return pl.pallas_call(
        functools.partial(_body, rows),
        grid=(n // rows,),
        in_specs=[
            pl.BlockSpec((n, fin), lambda i: (0, 0)),
            pl.BlockSpec((fin, hh), lambda i: (0, 0)),
            pl.BlockSpec((hh, 2), lambda i: (0, 0)),
            pl.BlockSpec((rows, n), lambda i: (i, 0)),
        ],
        out_specs=pl.BlockSpec((rows, n), lambda i: (i, 0)),
        out_shape=jax.ShapeDtypeStruct((n, n), h.dtype),
        scratch_shapes=[
            pltpu.VMEM((n, 1), jnp.float32),
            pltpu.VMEM((1, n), jnp.float32),
        ],
    )(h, W, a_cols, edge_mask)


# speedup vs baseline: 108.2805x; 1.1184x over previous
import functools

import jax
import jax.numpy as jnp
from jax.experimental import pallas as pl
from jax.experimental.pallas import tpu as pltpu


def _body(rows, h_ref, w_ref, ac_ref, m_ref, o_ref):
    i = pl.program_id(0)
    n = h_ref.shape[0]

    wa = jnp.dot(w_ref[...], ac_ref[...], preferred_element_type=jnp.float32)
    h_blk = h_ref[pl.ds(i * rows, rows), :]
    s1 = jnp.dot(h_blk, wa[:, 0:1],
                 preferred_element_type=jnp.float32)
    s2 = jnp.transpose(jnp.dot(h_ref[...], wa[:, 1:2],
                               preferred_element_type=jnp.float32))

    m = m_ref[...]
    z = s1 + s2
    z = jnp.maximum(z, 0.1 * z)
    e = jnp.where(m != 0.0, jnp.exp(z), 0.0)
    rs = jnp.sum(e, axis=1, keepdims=True)
    pos = rs == 0.0
    inv = jnp.where(pos, 1.0, 1.0 / rs)
    o_ref[...] = e * inv
    sub = o_ref[:, pl.ds(i * rows, rows)]
    r0 = jax.lax.broadcasted_iota(jnp.int32, (rows, rows), 0)
    c0 = jax.lax.broadcasted_iota(jnp.int32, (rows, rows), 1)
    o_ref[:, pl.ds(i * rows, rows)] = jnp.where((r0 == c0) & pos, 1.0, sub)


@jax.jit
def kernel(h, W, a, edge_mask):
    n, fin = h.shape
    hh = W.shape[1]
    rows = 256
    a_cols = a.reshape(2, hh).T

    return pl.pallas_call(
        functools.partial(_body, rows),
        grid=(n // rows,),
        in_specs=[
            pl.BlockSpec((n, fin), lambda i: (0, 0)),
            pl.BlockSpec((fin, hh), lambda i: (0, 0)),
            pl.BlockSpec((hh, 2), lambda i: (0, 0)),
            pl.BlockSpec((rows, n), lambda i: (i, 0)),
        ],
        out_specs=pl.BlockSpec((rows, n), lambda i: (i, 0)),
        out_shape=jax.ShapeDtypeStruct((n, n), h.dtype),
        compiler_params=pltpu.CompilerParams(
            dimension_semantics=("parallel",),
        ),
    )(h, W, a_cols, edge_mask)


# speedup vs baseline: 117.1722x; 1.0821x over previous
import functools

import jax
import jax.numpy as jnp
from jax.experimental import pallas as pl
from jax.experimental.pallas import tpu as pltpu


def _body(rows, h_ref, w_ref, ac_ref, m_ref, o_ref):
    i = pl.program_id(0)
    n = h_ref.shape[0]

    wa = jnp.dot(w_ref[...], ac_ref[...], preferred_element_type=jnp.float32)
    h_blk = h_ref[pl.ds(i * rows, rows), :]
    s1 = jnp.dot(h_blk, wa[:, 0:1],
                 preferred_element_type=jnp.float32)
    s2 = jnp.transpose(jnp.dot(h_ref[...], wa[:, 1:2],
                               preferred_element_type=jnp.float32))

    m = m_ref[...]
    z = s1 + s2
    z = jnp.maximum(z, 0.1 * z)
    e = jnp.where(m != 0.0, jnp.exp(z), 0.0)
    rs = jnp.sum(e, axis=1, keepdims=True)
    pos = rs == 0.0
    inv = jnp.where(pos, 1.0, 1.0 / rs)
    o_ref[...] = e * inv
    sub = o_ref[:, pl.ds(i * rows, rows)]
    r0 = jax.lax.broadcasted_iota(jnp.int32, (rows, rows), 0)
    c0 = jax.lax.broadcasted_iota(jnp.int32, (rows, rows), 1)
    o_ref[:, pl.ds(i * rows, rows)] = jnp.where((r0 == c0) & pos, 1.0, sub)


@jax.jit
def kernel(h, W, a, edge_mask):
    n, fin = h.shape
    hh = W.shape[1]
    rows = 512
    a_cols = a.reshape(2, hh).T

    return pl.pallas_call(
        functools.partial(_body, rows),
        grid=(n // rows,),
        in_specs=[
            pl.BlockSpec((n, fin), lambda i: (0, 0)),
            pl.BlockSpec((fin, hh), lambda i: (0, 0)),
            pl.BlockSpec((hh, 2), lambda i: (0, 0)),
            pl.BlockSpec((rows, n), lambda i: (i, 0)),
        ],
        out_specs=pl.BlockSpec((rows, n), lambda i: (i, 0)),
        out_shape=jax.ShapeDtypeStruct((n, n), h.dtype),
        compiler_params=pltpu.CompilerParams(
            dimension_semantics=("parallel",),
        ),
    )(h, W, a_cols, edge_mask)
